# baseline (speedup 1.0000x reference)
import jax
import jax.numpy as jnp
from jax.experimental import pallas as pl
from jax.experimental.pallas import tpu as pltpu

_B0 = 4
_N1 = 8
_ROWS = 16
_N_CHUNKS = _B0 * _N1
_SLOTS = 4
_LOOKAHEAD = 2


def _copy_body(x_ref, o_ref, buf, in_sems, out_sems):
    N, K, D = _N_CHUNKS, _SLOTS, _LOOKAHEAD

    def in_copy(c, slot):
        b, j = jax.lax.div(c, _N1), jax.lax.rem(c, _N1)
        src = x_ref.at[b, pl.ds(j * _ROWS, _ROWS)]
        return pltpu.make_async_copy(src, buf.at[slot], in_sems.at[slot])

    def out_copy(c, slot):
        b, j = jax.lax.div(c, _N1), jax.lax.rem(c, _N1)
        dst = o_ref.at[b, pl.ds(j * _ROWS, _ROWS)]
        return pltpu.make_async_copy(buf.at[slot], dst, out_sems.at[slot])

    for j in range(D):
        in_copy(j, j).start()

    def body(i, carry):
        slot = jax.lax.rem(i, K)
        in_copy(i, slot).wait()
        out_copy(i, slot).start()
        nxt = i + D

        @pl.when(nxt < N)
        def _():
            nslot = jax.lax.rem(nxt, K)

            @pl.when(nxt >= K)
            def _():
                out_copy(nxt - K, nslot).wait()

            in_copy(nxt, nslot).start()

        return carry

    jax.lax.fori_loop(0, N, body, 0)
    for c in range(N - K, N):
        out_copy(c, c % K).wait()


def kernel(inputs):
    return pl.pallas_call(
        _copy_body,
        out_shape=jax.ShapeDtypeStruct(inputs.shape, inputs.dtype),
        in_specs=[pl.BlockSpec(memory_space=pltpu.MemorySpace.HBM)],
        out_specs=pl.BlockSpec(memory_space=pltpu.MemorySpace.HBM),
        scratch_shapes=[
            pltpu.VMEM((_SLOTS, _ROWS, 2, 200, 176), jnp.float32),
            pltpu.SemaphoreType.DMA((_SLOTS,)),
            pltpu.SemaphoreType.DMA((_SLOTS,)),
        ],
    )(inputs)


# speedup vs baseline: 1.0003x; 1.0003x over previous
import jax
import jax.numpy as jnp
from jax.experimental import pallas as pl
from jax.experimental.pallas import tpu as pltpu

_B1 = 16


def _copy_body(x_ref, o_ref):
    o_ref[...] = x_ref[...]


def kernel(inputs):
    return pl.pallas_call(
        _copy_body,
        out_shape=jax.ShapeDtypeStruct(inputs.shape, inputs.dtype),
        grid=(4, 128 // _B1),
        in_specs=[pl.BlockSpec((1, _B1, 2, 200, 176), lambda b, j: (b, j, 0, 0, 0))],
        out_specs=pl.BlockSpec((1, _B1, 2, 200, 176), lambda b, j: (b, j, 0, 0, 0)),
    )(inputs)
